# baseline (speedup 1.0000x reference)
import functools

import jax
import jax.numpy as jnp
from jax import lax
from jax.experimental import pallas as pl
from jax.experimental.pallas import tpu as pltpu
from jax.experimental.pallas import tpu_sc as plsc


_NC = 2
_NS = 16
_NQ = 4
_C = 4096
_G = 128
_ZR = 32


def _make_spmm(n, d, nnz):
    rq = n // _NQ
    e_per_s = nnz // _NS
    n_chunks = e_per_s // _C
    rows_per_tile = rq // _NS
    assert rq * _NQ == n and e_per_s * _NS == nnz and n_chunks * _C == e_per_s
    assert rows_per_tile % _ZR == 0
    shift = rq.bit_length() - 1
    assert (1 << shift) == rq

    mesh = plsc.VectorSubcoreMesh(core_axis_name="c", subcore_axis_name="s")

    @functools.partial(
        pl.kernel,
        out_type=jax.ShapeDtypeStruct((n, d), jnp.float32),
        mesh=mesh,
        scratch_types=[
            pltpu.VMEM((2, _C), jnp.int32),
            pltpu.VMEM((2, _C), jnp.int32),
            pltpu.VMEM((2, _C), jnp.float32),
            pltpu.VMEM((_C,), jnp.int32),
            pltpu.VMEM((_C,), jnp.float32),
            pltpu.VMEM((_C,), jnp.int32),
            pltpu.VMEM((_G,), jnp.int32),
            pltpu.VMEM((_G,), jnp.int32),
            pltpu.VMEM((_G, d), jnp.bfloat16),
            pltpu.VMEM((_G, d), jnp.bfloat16),
            pltpu.VMEM((_G, d), jnp.float32),
            pltpu.VMEM((_G, d), jnp.float32),
            pltpu.VMEM((_ZR, d), jnp.float32),
            pltpu.VMEM_SHARED((rq, d), jnp.float32),
            pltpu.SemaphoreType.DMA,
            pltpu.SemaphoreType.DMA,
            pltpu.SemaphoreType.DMA,
            pltpu.SemaphoreType.DMA,
            pltpu.SemaphoreType.DMA,
            pltpu.SemaphoreType.DMA,
        ],
        compiler_params=pltpu.CompilerParams(
            needs_layout_passes=False, use_tc_tiling_on_sc=False),
    )
    def spmm(rows_hbm, cols_hbm, vals_hbm, x_hbm, z_hbm,
             rbuf, cbuf, vbuf, ccols, cvals, clrows, stage0, stage1,
             gbuf0, gbuf1, fbuf0, fbuf1, zbuf, acc, semi0, semi1,
             semg0, semg1, sems0, sems1):
        cid = lax.axis_index("c")
        sid = lax.axis_index("s")
        lane = lax.iota(jnp.int32, 16)
        zeros16 = jnp.zeros((16,), jnp.float32)
        semi = (semi0, semi1)
        semg = (semg0, semg1)
        sems = (sems0, sems1)
        stages = (stage0, stage1)
        gbufs = (gbuf0, gbuf1)
        fbufs = (fbuf0, fbuf1)

        def issue_inputs(ch, par):
            base = sid * e_per_s + ch * _C
            pltpu.async_copy(rows_hbm.at[pl.ds(base, _C)], rbuf.at[par],
                             semi[par])
            pltpu.async_copy(cols_hbm.at[pl.ds(base, _C)], cbuf.at[par],
                             semi[par])
            pltpu.async_copy(vals_hbm.at[pl.ds(base, _C)], vbuf.at[par],
                             semi[par])

        def wait_inputs(ch, par):
            base = sid * e_per_s + ch * _C
            pltpu.make_async_copy(rows_hbm.at[pl.ds(base, _C)], rbuf.at[par],
                                  semi[par]).wait()
            pltpu.make_async_copy(cols_hbm.at[pl.ds(base, _C)], cbuf.at[par],
                                  semi[par]).wait()
            pltpu.make_async_copy(vals_hbm.at[pl.ds(base, _C)], vbuf.at[par],
                                  semi[par]).wait()

        def issue_gather(b, gpar):
            pltpu.async_copy(x_hbm.at[ccols.at[pl.ds(b * _G, _G)]],
                             gbufs[gpar], semg[gpar])

        def wait_gather(b, gpar):
            pltpu.make_async_copy(x_hbm.at[ccols.at[pl.ds(b * _G, _G)]],
                                  gbufs[gpar], semg[gpar]).wait()

        def issue_scatter(gpar):
            pltpu.async_copy(fbufs[gpar], acc.at[stages[gpar]], sems[gpar],
                             add=True)

        def wait_scatter(gpar):
            pltpu.make_async_copy(fbufs[gpar], acc.at[stages[gpar]],
                                  sems[gpar]).wait()

        pad_cols = (cid * _NS + sid) * 16 + lane

        def prefill(i, _):
            sl = pl.ds(i * 16, 16)
            ccols[sl] = pad_cols
            clrows[sl] = lane
            cvals[sl] = zeros16
            return 0

        lax.fori_loop(0, _C // 16, prefill, 0)

        def zfill(i, _):
            for j in range(d // 16):
                zbuf[i, pl.ds(j * 16, 16)] = zeros16
            return 0

        lax.fori_loop(0, _ZR, zfill, 0)

        for p in range(_NQ // _NC):
            q = cid * (_NQ // _NC) + p

            for t in range(rows_per_tile // _ZR):
                pltpu.sync_copy(
                    zbuf, acc.at[pl.ds(sid * rows_per_tile + t * _ZR, _ZR)])
            plsc.subcore_barrier()

            def process_chunk(ch, par):
                wait_inputs(ch, par)

                @plsc.parallel_loop(0, _C // 16, unroll=4,
                                    carry=jnp.zeros((16,), jnp.int32))
                def off_vec(i, off):
                    sl = pl.ds(i * 16, 16)
                    r = rbuf[par, sl]
                    m = lax.shift_right_logical(r, shift) == q
                    inc = plsc.cumsum(m.astype(jnp.int32))
                    pos = off + inc - 1
                    plsc.store_scatter(ccols, [pos], cbuf[par, sl], mask=m)
                    plsc.store_scatter(cvals, [pos], vbuf[par, sl], mask=m)
                    plsc.store_scatter(
                        clrows, [pos], jnp.bitwise_and(r, rq - 1), mask=m)
                    return off + plsc.all_reduce_population_count(m)

                @pl.when(ch + 2 < n_chunks)
                def _():
                    issue_inputs(ch + 2, par)

                k = jnp.max(off_vec)
                nb = (k + _G - 1) // _G

                def tail(i, _):
                    idx = i * 16 + lane
                    plsc.store_scatter(cvals, [idx], zeros16, mask=idx >= k)
                    return 0

                lax.fori_loop(k // 16, nb * (_G // 16), tail, 0)

                @pl.when(nb > 0)
                def _():
                    issue_gather(0, 0)

                def do_sub(b, gpar):
                    @pl.when(b >= 1)
                    def _():
                        wait_scatter(1 - gpar)

                    @pl.when(b + 1 < nb)
                    def _():
                        issue_gather(b + 1, 1 - gpar)

                    gbuf = gbufs[gpar]
                    fbuf = fbufs[gpar]
                    stage = stages[gpar]
                    b0 = b * _G
                    for j in range(_G // 16):
                        stage[pl.ds(j * 16, 16)] = clrows[pl.ds(b0 + j * 16, 16)]
                    wait_gather(b, gpar)

                    @plsc.parallel_loop(0, _G // 16, unroll=2)
                    def _(g2):
                        vv = cvals[pl.ds(b0 + g2 * 16, 16)]
                        for e in range(16):
                            vb = jnp.take_along_axis(
                                vv, jnp.full((16,), e, jnp.int32), axis=0)
                            le = g2 * 16 + e
                            for h in range(d // 32):
                                v32 = gbuf[le, pl.ds(h * 32, 32)]
                                lo, hi = plsc.unpack(
                                    v32, format=plsc.PackFormat.INTERLEAVED)
                                fbuf[le, pl.ds(h * 32, 16)] = lo * vb
                                fbuf[le, pl.ds(h * 32 + 16, 16)] = hi * vb

                    issue_scatter(gpar)

                def sub2(jb, _):
                    b = 2 * jb

                    @pl.when(b < nb)
                    def _():
                        do_sub(b, 0)

                    @pl.when(b + 1 < nb)
                    def _():
                        do_sub(b + 1, 1)

                    return 0

                lax.fori_loop(0, (nb + 1) // 2, sub2, 0)

                @pl.when(nb >= 1)
                def _():
                    @pl.when(jnp.bitwise_and(nb - 1, 1) == 0)
                    def _():
                        wait_scatter(0)

                    @pl.when(jnp.bitwise_and(nb - 1, 1) == 1)
                    def _():
                        wait_scatter(1)

            issue_inputs(0, 0)
            issue_inputs(1, 1)

            def chunk2_body(j, _):
                process_chunk(2 * j, 0)
                process_chunk(2 * j + 1, 1)
                return 0

            lax.fori_loop(0, n_chunks // 2, chunk2_body, 0)
            plsc.subcore_barrier()

            row0 = q * rq + sid * rows_per_tile
            pltpu.sync_copy(acc.at[pl.ds(sid * rows_per_tile, rows_per_tile)],
                            z_hbm.at[pl.ds(row0, rows_per_tile)])

    return spmm


_RB = 2048


def _dense_relu_kernel(z_ref, w_ref, y_ref):
    z = z_ref[...]
    y = jnp.dot(z, w_ref[...].T, preferred_element_type=jnp.float32)
    y_ref[...] = jnp.maximum(y, 0.0)


def _dense_relu(z, w):
    n, d = z.shape
    return pl.pallas_call(
        _dense_relu_kernel,
        grid=(n // _RB,),
        in_specs=[
            pl.BlockSpec((_RB, d), lambda i: (i, 0)),
            pl.BlockSpec((d, d), lambda i: (0, 0)),
        ],
        out_specs=pl.BlockSpec((_RB, d), lambda i: (i, 0)),
        out_shape=jax.ShapeDtypeStruct((n, d), jnp.float32),
    )(z, w)


def _l2n(x):
    nrm = jnp.sqrt(jnp.sum(x * x, axis=-1, keepdims=True))
    return x / jnp.maximum(nrm, 1e-12)


def _final_kernel(emb_ref, y1_ref, z2_ref, w_ref, out_ref):
    y2 = jnp.maximum(
        jnp.dot(z2_ref[...], w_ref[...].T,
                preferred_element_type=jnp.float32), 0.0)
    out_ref[...] = (_l2n(emb_ref[...]) + _l2n(y1_ref[...]) + _l2n(y2)) / 3.0


def _final(emb, y1, z2, w1):
    n, d = emb.shape
    return pl.pallas_call(
        _final_kernel,
        grid=(n // _RB,),
        in_specs=[
            pl.BlockSpec((_RB, d), lambda i: (i, 0)),
            pl.BlockSpec((_RB, d), lambda i: (i, 0)),
            pl.BlockSpec((_RB, d), lambda i: (i, 0)),
            pl.BlockSpec((d, d), lambda i: (0, 0)),
        ],
        out_specs=pl.BlockSpec((_RB, d), lambda i: (i, 0)),
        out_shape=jax.ShapeDtypeStruct((n, d), jnp.float32),
    )(emb, y1, z2, w1)


def _bf16_perm(x):
    n, d = x.shape
    return (x.reshape(n, d // 32, 2, 16).transpose(0, 1, 3, 2)
            .reshape(n, d).astype(jnp.bfloat16))


def kernel(hyper_indices, hyper_values, embedding, W0, W1):
    n, d = embedding.shape
    nnz = hyper_values.shape[0]
    rows = hyper_indices[0]
    cols = hyper_indices[1]
    spmm = _make_spmm(n, d, nnz)

    z1 = spmm(rows, cols, hyper_values, _bf16_perm(embedding))
    y1 = _dense_relu(z1, W0)
    z2 = spmm(rows, cols, hyper_values, _bf16_perm(y1))
    return _final(embedding, y1, z2, W1)


# speedup vs baseline: 1.2288x; 1.2288x over previous
import functools

import jax
import jax.numpy as jnp
from jax import lax
from jax.experimental import pallas as pl
from jax.experimental.pallas import tpu as pltpu
from jax.experimental.pallas import tpu_sc as plsc


_NC = 2
_NS = 16
_NQ = 4
_C = 2048
_G = 128
_GSH = 7
_ZR = 64


def _make_spmm(n, d, nnz):
    rq = n // _NQ
    e_per_s = nnz // _NS
    n_chunks = e_per_s // _C
    rows_per_tile = rq // _NS
    assert rq * _NQ == n and e_per_s * _NS == nnz and n_chunks * _C == e_per_s
    assert rows_per_tile % _ZR == 0
    shift = rq.bit_length() - 1
    assert (1 << shift) == rq

    mesh = plsc.VectorSubcoreMesh(core_axis_name="c", subcore_axis_name="s")

    @functools.partial(
        pl.kernel,
        out_type=jax.ShapeDtypeStruct((n, d), jnp.float32),
        mesh=mesh,
        scratch_types=[
            pltpu.VMEM((2, _C), jnp.int32),
            pltpu.VMEM((2, _C), jnp.int32),
            pltpu.VMEM((2, _C), jnp.float32),
            pltpu.VMEM((_C,), jnp.int32),
            pltpu.VMEM((_C,), jnp.float32),
            pltpu.VMEM((_C // _G, _G), jnp.int32),
            pltpu.VMEM((_G, d), jnp.float32),
            pltpu.VMEM((_G, d), jnp.float32),
            pltpu.VMEM((_G, d), jnp.float32),
            pltpu.VMEM((_ZR, d), jnp.float32),
            pltpu.VMEM_SHARED((rq, d), jnp.float32),
            pltpu.SemaphoreType.DMA,
            pltpu.SemaphoreType.DMA,
            pltpu.SemaphoreType.DMA,
            pltpu.SemaphoreType.DMA,
            pltpu.SemaphoreType.DMA,
            pltpu.SemaphoreType.DMA,
            pltpu.SemaphoreType.DMA,
            pltpu.SemaphoreType.DMA,
        ],
        compiler_params=pltpu.CompilerParams(
            needs_layout_passes=False, use_tc_tiling_on_sc=False),
    )
    def spmm(rows_hbm, cols_hbm, vals_hbm, x_hbm, z_hbm,
             rbuf, cbuf, vbuf, ccols, cvals, stage, gbuf0, gbuf1, gbuf2,
             zbuf, acc, semi0, semi1, semg0, semg1, semg2,
             sems0, sems1, sems2):
        cid = lax.axis_index("c")
        sid = lax.axis_index("s")
        lane = lax.iota(jnp.int32, 16)
        zeros16 = jnp.zeros((16,), jnp.float32)
        semi = (semi0, semi1)
        semg = (semg0, semg1, semg2)
        sems = (sems0, sems1, sems2)
        gbufs = (gbuf0, gbuf1, gbuf2)

        def issue_inputs(ch, par):
            base = sid * e_per_s + ch * _C
            pltpu.async_copy(rows_hbm.at[pl.ds(base, _C)], rbuf.at[par],
                             semi[par])
            pltpu.async_copy(cols_hbm.at[pl.ds(base, _C)], cbuf.at[par],
                             semi[par])
            pltpu.async_copy(vals_hbm.at[pl.ds(base, _C)], vbuf.at[par],
                             semi[par])

        def wait_inputs(ch, par):
            base = sid * e_per_s + ch * _C
            pltpu.make_async_copy(rows_hbm.at[pl.ds(base, _C)], rbuf.at[par],
                                  semi[par]).wait()
            pltpu.make_async_copy(cols_hbm.at[pl.ds(base, _C)], cbuf.at[par],
                                  semi[par]).wait()
            pltpu.make_async_copy(vals_hbm.at[pl.ds(base, _C)], vbuf.at[par],
                                  semi[par]).wait()

        def issue_gather(b, gpar):
            pltpu.async_copy(x_hbm.at[ccols.at[pl.ds(b * _G, _G)]],
                             gbufs[gpar], semg[gpar])

        def wait_gather(b, gpar):
            pltpu.make_async_copy(x_hbm.at[ccols.at[pl.ds(b * _G, _G)]],
                                  gbufs[gpar], semg[gpar]).wait()

        def issue_scatter(b, gpar):
            pltpu.async_copy(gbufs[gpar], acc.at[stage.at[b]], sems[gpar],
                             add=True)

        def wait_scatter(gpar):
            pltpu.make_async_copy(gbufs[gpar], acc.at[stage.at[0]],
                                  sems[gpar]).wait()

        pad_cols = (cid * _NS + sid) * 16 + lane

        def prefill(i, _):
            sl = pl.ds(i * 16, 16)
            ccols[sl] = pad_cols
            cvals[sl] = zeros16
            return 0

        lax.fori_loop(0, _C // 16, prefill, 0)

        for sr in range(_C // _G):
            for sj in range(_G // 16):
                stage[sr, pl.ds(sj * 16, 16)] = lane

        def zfill(i, _):
            for j in range(d // 16):
                zbuf[i, pl.ds(j * 16, 16)] = zeros16
            return 0

        lax.fori_loop(0, _ZR, zfill, 0)

        for p in range(_NQ // _NC):
            q = cid * (_NQ // _NC) + p

            for t in range(rows_per_tile // _ZR):
                pltpu.sync_copy(
                    zbuf, acc.at[pl.ds(sid * rows_per_tile + t * _ZR, _ZR)])
            plsc.subcore_barrier()

            def process_chunk(ch, par):
                wait_inputs(ch, par)

                @plsc.parallel_loop(0, _C // 16, unroll=8,
                                    carry=jnp.zeros((16,), jnp.int32))
                def off_vec(i, off):
                    sl = pl.ds(i * 16, 16)
                    r = rbuf[par, sl]
                    m = lax.shift_right_logical(r, shift) == q
                    inc = plsc.cumsum(m.astype(jnp.int32))
                    pos = off + inc - 1
                    plsc.store_scatter(ccols, [pos], cbuf[par, sl], mask=m)
                    plsc.store_scatter(cvals, [pos], vbuf[par, sl], mask=m)
                    plsc.store_scatter(
                        stage,
                        [lax.shift_right_logical(pos, _GSH),
                         jnp.bitwise_and(pos, _G - 1)],
                        jnp.bitwise_and(r, rq - 1), mask=m)
                    return off + plsc.all_reduce_population_count(m)

                @pl.when(ch + 2 < n_chunks)
                def _():
                    issue_inputs(ch + 2, par)

                k = jnp.max(off_vec)
                nb = (k + _G - 1) // _G

                def tail(i, _):
                    idx = i * 16 + lane
                    plsc.store_scatter(cvals, [idx], zeros16, mask=idx >= k)
                    return 0

                lax.fori_loop(k // 16, nb * (_G // 16), tail, 0)

                @pl.when(nb > 0)
                def _():
                    issue_gather(0, 0)

                @pl.when(nb > 1)
                def _():
                    issue_gather(1, 1)

                def do_sub(b, gpar):
                    @pl.when(b >= 1)
                    def _():
                        wait_scatter((gpar + 2) % 3)

                    @pl.when(b + 2 < nb)
                    def _():
                        issue_gather(b + 2, (gpar + 2) % 3)

                    gbuf = gbufs[gpar]
                    b0 = b * _G
                    wait_gather(b, gpar)

                    @plsc.parallel_loop(0, _G // 16, unroll=2)
                    def _(g2):
                        vv = cvals[pl.ds(b0 + g2 * 16, 16)]
                        for e in range(16):
                            vb = jnp.take_along_axis(
                                vv, jnp.full((16,), e, jnp.int32), axis=0)
                            le = g2 * 16 + e
                            for j in range(d // 16):
                                sl = pl.ds(j * 16, 16)
                                gbuf[le, sl] = gbuf[le, sl] * vb

                    issue_scatter(b, gpar)

                def sub3(jb, _):
                    b = 3 * jb

                    @pl.when(b < nb)
                    def _():
                        do_sub(b, 0)

                    @pl.when(b + 1 < nb)
                    def _():
                        do_sub(b + 1, 1)

                    @pl.when(b + 2 < nb)
                    def _():
                        do_sub(b + 2, 2)

                    return 0

                lax.fori_loop(0, (nb + 2) // 3, sub3, 0)

                @pl.when(nb >= 1)
                def _():
                    last = lax.rem(nb - 1, 3)
                    for t in range(3):
                        @pl.when(last == t)
                        def _(t=t):
                            wait_scatter(t)

            issue_inputs(0, 0)
            issue_inputs(1, 1)

            def chunk2_body(j, _):
                process_chunk(2 * j, 0)
                process_chunk(2 * j + 1, 1)
                return 0

            lax.fori_loop(0, n_chunks // 2, chunk2_body, 0)
            plsc.subcore_barrier()

            row0 = q * rq + sid * rows_per_tile
            pltpu.sync_copy(acc.at[pl.ds(sid * rows_per_tile, rows_per_tile)],
                            z_hbm.at[pl.ds(row0, rows_per_tile)])

    return spmm


_RB = 2048


def _dense_relu_kernel(z_ref, w_ref, y_ref):
    z = z_ref[...]
    y = jnp.dot(z, w_ref[...].T, preferred_element_type=jnp.float32)
    y_ref[...] = jnp.maximum(y, 0.0)


def _dense_relu(z, w):
    n, d = z.shape
    return pl.pallas_call(
        _dense_relu_kernel,
        grid=(n // _RB,),
        in_specs=[
            pl.BlockSpec((_RB, d), lambda i: (i, 0)),
            pl.BlockSpec((d, d), lambda i: (0, 0)),
        ],
        out_specs=pl.BlockSpec((_RB, d), lambda i: (i, 0)),
        out_shape=jax.ShapeDtypeStruct((n, d), jnp.float32),
    )(z, w)


def _l2n(x):
    nrm = jnp.sqrt(jnp.sum(x * x, axis=-1, keepdims=True))
    return x / jnp.maximum(nrm, 1e-12)


def _final_kernel(emb_ref, y1_ref, z2_ref, w_ref, out_ref):
    y2 = jnp.maximum(
        jnp.dot(z2_ref[...], w_ref[...].T,
                preferred_element_type=jnp.float32), 0.0)
    out_ref[...] = (_l2n(emb_ref[...]) + _l2n(y1_ref[...]) + _l2n(y2)) / 3.0


def _final(emb, y1, z2, w1):
    n, d = emb.shape
    return pl.pallas_call(
        _final_kernel,
        grid=(n // _RB,),
        in_specs=[
            pl.BlockSpec((_RB, d), lambda i: (i, 0)),
            pl.BlockSpec((_RB, d), lambda i: (i, 0)),
            pl.BlockSpec((_RB, d), lambda i: (i, 0)),
            pl.BlockSpec((d, d), lambda i: (0, 0)),
        ],
        out_specs=pl.BlockSpec((_RB, d), lambda i: (i, 0)),
        out_shape=jax.ShapeDtypeStruct((n, d), jnp.float32),
    )(emb, y1, z2, w1)


def kernel(hyper_indices, hyper_values, embedding, W0, W1):
    n, d = embedding.shape
    nnz = hyper_values.shape[0]
    rows = hyper_indices[0]
    cols = hyper_indices[1]
    spmm = _make_spmm(n, d, nnz)

    z1 = spmm(rows, cols, hyper_values, embedding)
    y1 = _dense_relu(z1, W0)
    z2 = spmm(rows, cols, hyper_values, y1)
    return _final(embedding, y1, z2, W1)


# speedup vs baseline: 1.4538x; 1.1831x over previous
import functools

import jax
import jax.numpy as jnp
from jax import lax
from jax.experimental import pallas as pl
from jax.experimental.pallas import tpu as pltpu
from jax.experimental.pallas import tpu_sc as plsc


_NC = 2
_NS = 16
_NQ = 4
_C = 4096
_G = 128
_GSH = 7
_ZR = 64


def _make_spmm(n, d, nnz):
    rq = n // _NQ
    e_per_s = nnz // _NS
    n_chunks = e_per_s // _C
    rows_per_tile = rq // _NS
    assert rq * _NQ == n and e_per_s * _NS == nnz and n_chunks * _C == e_per_s
    assert rows_per_tile % _ZR == 0
    shift = rq.bit_length() - 1
    assert (1 << shift) == rq

    mesh = plsc.VectorSubcoreMesh(core_axis_name="c", subcore_axis_name="s")

    @functools.partial(
        pl.kernel,
        out_type=jax.ShapeDtypeStruct((n, d), jnp.float32),
        mesh=mesh,
        scratch_types=[
            pltpu.VMEM((2, _C), jnp.int32),
            pltpu.VMEM((2, _C), jnp.int32),
            pltpu.VMEM((2, _C), jnp.float32),
            pltpu.VMEM((_C,), jnp.int32),
            pltpu.VMEM((_C,), jnp.float32),
            pltpu.VMEM((_C // _G, _G), jnp.int32),
            pltpu.VMEM((_G, d), jnp.float32),
            pltpu.VMEM((_G, d), jnp.float32),
            pltpu.VMEM((_G, d), jnp.float32),
            pltpu.VMEM((_ZR, d), jnp.float32),
            pltpu.VMEM_SHARED((rq, d), jnp.float32),
            pltpu.SemaphoreType.DMA,
            pltpu.SemaphoreType.DMA,
            pltpu.SemaphoreType.DMA,
            pltpu.SemaphoreType.DMA,
            pltpu.SemaphoreType.DMA,
            pltpu.SemaphoreType.DMA,
            pltpu.SemaphoreType.DMA,
            pltpu.SemaphoreType.DMA,
        ],
        compiler_params=pltpu.CompilerParams(
            needs_layout_passes=False, use_tc_tiling_on_sc=False),
    )
    def spmm(rows_hbm, cols_hbm, vals_hbm, x_hbm, z_hbm,
             rbuf, cbuf, vbuf, ccols, cvals, stage, gbuf0, gbuf1, gbuf2,
             zbuf, acc, semi0, semi1, semg0, semg1, semg2,
             sems0, sems1, sems2):
        cid = lax.axis_index("c")
        sid = lax.axis_index("s")
        lane = lax.iota(jnp.int32, 16)
        zeros16 = jnp.zeros((16,), jnp.float32)
        semi = (semi0, semi1)
        semg = (semg0, semg1, semg2)
        sems = (sems0, sems1, sems2)
        gbufs = (gbuf0, gbuf1, gbuf2)

        def issue_inputs(ch, par):
            base = sid * e_per_s + ch * _C
            pltpu.async_copy(rows_hbm.at[pl.ds(base, _C)], rbuf.at[par],
                             semi[par])
            pltpu.async_copy(cols_hbm.at[pl.ds(base, _C)], cbuf.at[par],
                             semi[par])
            pltpu.async_copy(vals_hbm.at[pl.ds(base, _C)], vbuf.at[par],
                             semi[par])

        def wait_inputs(ch, par):
            base = sid * e_per_s + ch * _C
            pltpu.make_async_copy(rows_hbm.at[pl.ds(base, _C)], rbuf.at[par],
                                  semi[par]).wait()
            pltpu.make_async_copy(cols_hbm.at[pl.ds(base, _C)], cbuf.at[par],
                                  semi[par]).wait()
            pltpu.make_async_copy(vals_hbm.at[pl.ds(base, _C)], vbuf.at[par],
                                  semi[par]).wait()

        def issue_gather(b, gpar):
            pltpu.async_copy(x_hbm.at[ccols.at[pl.ds(b * _G, _G)]],
                             gbufs[gpar], semg[gpar])

        def wait_gather(b, gpar):
            pltpu.make_async_copy(x_hbm.at[ccols.at[pl.ds(b * _G, _G)]],
                                  gbufs[gpar], semg[gpar]).wait()

        def issue_scatter(b, gpar):
            pltpu.async_copy(gbufs[gpar], acc.at[stage.at[b]], sems[gpar],
                             add=True)

        def wait_scatter(gpar):
            pltpu.make_async_copy(gbufs[gpar], acc.at[stage.at[0]],
                                  sems[gpar]).wait()

        pad_cols = (cid * _NS + sid) * 16 + lane

        def prefill(i, _):
            sl = pl.ds(i * 16, 16)
            ccols[sl] = pad_cols
            cvals[sl] = zeros16
            return 0

        lax.fori_loop(0, _C // 16, prefill, 0)

        for sr in range(_C // _G):
            for sj in range(_G // 16):
                stage[sr, pl.ds(sj * 16, 16)] = lane

        def zfill(i, _):
            for j in range(d // 16):
                zbuf[i, pl.ds(j * 16, 16)] = zeros16
            return 0

        lax.fori_loop(0, _ZR, zfill, 0)

        for p in range(_NQ // _NC):
            q = cid * (_NQ // _NC) + p

            for t in range(rows_per_tile // _ZR):
                pltpu.sync_copy(
                    zbuf, acc.at[pl.ds(sid * rows_per_tile + t * _ZR, _ZR)])
            plsc.subcore_barrier()

            def process_chunk(ch, par):
                wait_inputs(ch, par)

                @plsc.parallel_loop(0, _C // 16, unroll=8,
                                    carry=jnp.zeros((16,), jnp.int32))
                def off_vec(i, off):
                    sl = pl.ds(i * 16, 16)
                    r = rbuf[par, sl]
                    m = lax.shift_right_logical(r, shift) == q
                    inc = plsc.cumsum(m.astype(jnp.int32))
                    pos = off + inc - 1
                    plsc.store_scatter(ccols, [pos], cbuf[par, sl], mask=m)
                    plsc.store_scatter(cvals, [pos], vbuf[par, sl], mask=m)
                    plsc.store_scatter(
                        stage,
                        [lax.shift_right_logical(pos, _GSH),
                         jnp.bitwise_and(pos, _G - 1)],
                        jnp.bitwise_and(r, rq - 1), mask=m)
                    return off + plsc.all_reduce_population_count(m)

                @pl.when(ch + 2 < n_chunks)
                def _():
                    issue_inputs(ch + 2, par)

                k = jnp.max(off_vec)
                nb = (k + _G - 1) // _G

                def tail(i, _):
                    idx = i * 16 + lane
                    plsc.store_scatter(cvals, [idx], zeros16, mask=idx >= k)
                    return 0

                lax.fori_loop(k // 16, nb * (_G // 16), tail, 0)

                @pl.when(nb > 0)
                def _():
                    issue_gather(0, 0)

                @pl.when(nb > 1)
                def _():
                    issue_gather(1, 1)

                def do_sub(b, gpar):
                    @pl.when(b >= 1)
                    def _():
                        wait_scatter((gpar + 2) % 3)

                    @pl.when(b + 2 < nb)
                    def _():
                        issue_gather(b + 2, (gpar + 2) % 3)

                    gbuf = gbufs[gpar]
                    b0 = b * _G
                    wait_gather(b, gpar)

                    @plsc.parallel_loop(0, _G // 16, unroll=2)
                    def _(g2):
                        vv = cvals[pl.ds(b0 + g2 * 16, 16)]
                        for e in range(16):
                            vb = jnp.take_along_axis(
                                vv, jnp.full((16,), e, jnp.int32), axis=0)
                            le = g2 * 16 + e
                            for j in range(d // 16):
                                sl = pl.ds(j * 16, 16)
                                gbuf[le, sl] = gbuf[le, sl] * vb

                    issue_scatter(b, gpar)

                def sub3(jb, _):
                    b = 3 * jb

                    @pl.when(b < nb)
                    def _():
                        do_sub(b, 0)

                    @pl.when(b + 1 < nb)
                    def _():
                        do_sub(b + 1, 1)

                    @pl.when(b + 2 < nb)
                    def _():
                        do_sub(b + 2, 2)

                    return 0

                lax.fori_loop(0, (nb + 2) // 3, sub3, 0)

                @pl.when(nb >= 1)
                def _():
                    last = lax.rem(nb - 1, 3)
                    for t in range(3):
                        @pl.when(last == t)
                        def _(t=t):
                            wait_scatter(t)

            issue_inputs(0, 0)
            issue_inputs(1, 1)

            def chunk2_body(j, _):
                process_chunk(2 * j, 0)
                process_chunk(2 * j + 1, 1)
                return 0

            lax.fori_loop(0, n_chunks // 2, chunk2_body, 0)
            plsc.subcore_barrier()

            row0 = q * rq + sid * rows_per_tile
            pltpu.sync_copy(acc.at[pl.ds(sid * rows_per_tile, rows_per_tile)],
                            z_hbm.at[pl.ds(row0, rows_per_tile)])

    return spmm


_RB = 2048


def _dense_relu_kernel(z_ref, w_ref, y_ref):
    z = z_ref[...]
    y = jnp.dot(z, w_ref[...].T, preferred_element_type=jnp.float32)
    y_ref[...] = jnp.maximum(y, 0.0)


def _dense_relu(z, w):
    n, d = z.shape
    return pl.pallas_call(
        _dense_relu_kernel,
        grid=(n // _RB,),
        in_specs=[
            pl.BlockSpec((_RB, d), lambda i: (i, 0)),
            pl.BlockSpec((d, d), lambda i: (0, 0)),
        ],
        out_specs=pl.BlockSpec((_RB, d), lambda i: (i, 0)),
        out_shape=jax.ShapeDtypeStruct((n, d), jnp.float32),
    )(z, w)


def _l2n(x):
    nrm = jnp.sqrt(jnp.sum(x * x, axis=-1, keepdims=True))
    return x / jnp.maximum(nrm, 1e-12)


def _final_kernel(emb_ref, y1_ref, z2_ref, w_ref, out_ref):
    y2 = jnp.maximum(
        jnp.dot(z2_ref[...], w_ref[...].T,
                preferred_element_type=jnp.float32), 0.0)
    out_ref[...] = (_l2n(emb_ref[...]) + _l2n(y1_ref[...]) + _l2n(y2)) / 3.0


def _final(emb, y1, z2, w1):
    n, d = emb.shape
    return pl.pallas_call(
        _final_kernel,
        grid=(n // _RB,),
        in_specs=[
            pl.BlockSpec((_RB, d), lambda i: (i, 0)),
            pl.BlockSpec((_RB, d), lambda i: (i, 0)),
            pl.BlockSpec((_RB, d), lambda i: (i, 0)),
            pl.BlockSpec((d, d), lambda i: (0, 0)),
        ],
        out_specs=pl.BlockSpec((_RB, d), lambda i: (i, 0)),
        out_shape=jax.ShapeDtypeStruct((n, d), jnp.float32),
    )(emb, y1, z2, w1)


def kernel(hyper_indices, hyper_values, embedding, W0, W1):
    n, d = embedding.shape
    nnz = hyper_values.shape[0]
    rows = hyper_indices[0]
    cols = hyper_indices[1]
    spmm = _make_spmm(n, d, nnz)

    z1 = spmm(rows, cols, hyper_values, embedding)
    y1 = _dense_relu(z1, W0)
    z2 = spmm(rows, cols, hyper_values, y1)
    return _final(embedding, y1, z2, W1)
